# baseline (speedup 1.0000x reference)
import functools

import jax
import jax.numpy as jnp
from jax import lax
from jax.experimental import pallas as pl
from jax.experimental.pallas import tpu as pltpu
from jax.experimental.pallas import tpu_sc as plsc

_TOP_K = 2
_HID = 1024
_BM = 512


def _gating_kernel(x_ref, gw_ref, gb_ref, gs_ref, acc_ref):
    i = pl.program_id(0)
    n = pl.num_programs(0)
    num_experts = gw_ref.shape[0]
    logits = jax.lax.dot_general(
        x_ref[...], gw_ref[...],
        dimension_numbers=(((1,), (1,)), ((), ())),
        preferred_element_type=jnp.float32,
    )
    part = jnp.sum(logits, axis=0, keepdims=True)

    @pl.when(i == 0)
    def _():
        acc_ref[:1, :num_experts] = part

    @pl.when(i > 0)
    def _():
        acc_ref[:1, :num_experts] += part

    @pl.when(i == n - 1)
    def _():
        gs_ref[...] = acc_ref[:1, :num_experts] + gb_ref[...]


def _sc_rank_experts(num_experts):
    mesh = plsc.VectorSubcoreMesh(core_axis_name="c", subcore_axis_name="s")

    @functools.partial(
        pl.kernel,
        mesh=mesh,
        out_type=jax.ShapeDtypeStruct((num_experts,), jnp.int32),
        scratch_types=[
            pltpu.VMEM((num_experts,), jnp.float32),
            pltpu.VMEM((num_experts,), jnp.int32),
        ],
    )
    def rank(gs_hbm, order_hbm, gs_v, iv_v):
        c = lax.axis_index("c")
        sub = lax.axis_index("s")

        @pl.when((c == 0) & (sub == 0))
        def _():
            pltpu.sync_copy(gs_hbm, gs_v)
            iota = lax.iota(jnp.int32, num_experts)
            keys = gs_v[...]
            best = keys[0]
            besti = jnp.int32(0)
            sec = keys[1]
            seci = jnp.int32(1)
            swap = sec > best
            best, sec = jnp.where(swap, sec, best), jnp.where(swap, best, sec)
            besti, seci = (jnp.where(swap, seci, besti),
                           jnp.where(swap, besti, seci))
            for i in range(2, num_experts):
                v = keys[i]
                gt = v > best
                mid = v > sec
                nbest = jnp.where(gt, v, best)
                nbesti = jnp.where(gt, jnp.int32(i), besti)
                nsec = jnp.where(gt, best, jnp.where(mid, v, sec))
                nseci = jnp.where(gt, besti, jnp.where(mid, jnp.int32(i), seci))
                best, besti, sec, seci = nbest, nbesti, nsec, nseci
            iv_v[...] = jnp.where(iota == 0, besti,
                                  jnp.where(iota == 1, seci, jnp.int32(0)))
            pltpu.sync_copy(iv_v, order_hbm)

    return rank


def _collapse_kernel(idx_ref, w_ref, a_ref, m0_ref, m1_ref, wb_ref):
    k = pl.program_id(0)
    j = pl.program_id(1)

    @pl.when(j == 0)
    def _():
        wb_ref[...] = w_ref[...].astype(jnp.bfloat16)

    p = jax.lax.dot_general(
        wb_ref[...], a_ref[...].astype(jnp.bfloat16),
        dimension_numbers=(((1,), (1,)), ((), ())),
        preferred_element_type=jnp.float32,
    )

    @pl.when((k == 0) & (j == 0))
    def _():
        m0_ref[...] = p.astype(jnp.bfloat16)

    @pl.when((k == 0) & (j > 0))
    def _():
        m1_ref[...] = p.astype(jnp.bfloat16)

    @pl.when((k > 0) & (j == 0))
    def _():
        m0_ref[...] = (m0_ref[...].astype(jnp.float32) + p).astype(jnp.bfloat16)

    @pl.when((k > 0) & (j > 0))
    def _():
        m1_ref[...] = (m1_ref[...].astype(jnp.float32) + p).astype(jnp.bfloat16)


def _moe_kernel(x_ref, m0_ref, m1_ref, b_ref, o_ref):
    xb = x_ref[...].astype(jnp.bfloat16)
    hn = m0_ref.shape[1]
    y0 = jax.lax.dot_general(
        xb, m0_ref[...],
        dimension_numbers=(((1,), (0,)), ((), ())),
        preferred_element_type=jnp.float32,
    )
    y1 = jax.lax.dot_general(
        xb, m1_ref[...],
        dimension_numbers=(((1,), (0,)), ((), ())),
        preferred_element_type=jnp.float32,
    )
    o_ref[:, :hn] = y0 + b_ref[:, :hn]
    o_ref[:, hn:] = y1 + b_ref[:, hn:]


def kernel(x, gating_w, gating_b, weight, agg_w, agg_b):
    batch, in_dim = x.shape
    num_experts = gating_w.shape[0]
    out_dim = agg_w.shape[0]

    gb_total = (gating_b.astype(jnp.float32) * batch).reshape(1, num_experts)

    bm_gate = 1024
    gs = pl.pallas_call(
        _gating_kernel,
        grid=(batch // bm_gate,),
        in_specs=[
            pl.BlockSpec((bm_gate, in_dim), lambda i: (i, 0)),
            pl.BlockSpec((num_experts, in_dim), lambda i: (0, 0)),
            pl.BlockSpec((1, num_experts), lambda i: (0, 0)),
        ],
        out_specs=pl.BlockSpec((1, num_experts), lambda i: (0, 0)),
        out_shape=jax.ShapeDtypeStruct((1, num_experts), jnp.float32),
        scratch_shapes=[pltpu.VMEM((8, 128), jnp.float32)],
    )(x, gating_w, gb_total)

    idx = _sc_rank_experts(num_experts)(gs.reshape(num_experts))

    bn = out_dim // 2
    collapse_spec = pltpu.PrefetchScalarGridSpec(
        num_scalar_prefetch=1,
        grid=(_TOP_K, out_dim // bn),
        in_specs=[
            pl.BlockSpec((in_dim, _HID), lambda k, j, idx_ref: (0, idx_ref[k])),
            pl.BlockSpec((bn, _HID), lambda k, j, idx_ref: (j, idx_ref[k])),
        ],
        out_specs=[
            pl.BlockSpec((in_dim, bn), lambda k, j, idx_ref: (0, 0)),
            pl.BlockSpec((in_dim, bn), lambda k, j, idx_ref: (0, 0)),
        ],
        scratch_shapes=[
            pltpu.VMEM((in_dim, _HID), jnp.bfloat16),
        ],
    )
    m0, m1 = pl.pallas_call(
        _collapse_kernel,
        grid_spec=collapse_spec,
        out_shape=[
            jax.ShapeDtypeStruct((in_dim, bn), jnp.bfloat16),
            jax.ShapeDtypeStruct((in_dim, bn), jnp.bfloat16),
        ],
        compiler_params=pltpu.CompilerParams(
            vmem_limit_bytes=63 * 1024 * 1024,
        ),
    )(idx, weight, agg_w)


    b2 = agg_b.reshape(1, out_dim)
    out = pl.pallas_call(
        _moe_kernel,
        grid=(batch // _BM,),
        in_specs=[
            pl.BlockSpec((_BM, in_dim), lambda i: (i, 0)),
            pl.BlockSpec((in_dim, bn), lambda i: (0, 0)),
            pl.BlockSpec((in_dim, bn), lambda i: (0, 0)),
            pl.BlockSpec((1, out_dim), lambda i: (0, 0)),
        ],
        out_specs=pl.BlockSpec((_BM, out_dim), lambda i: (i, 0)),
        out_shape=jax.ShapeDtypeStruct((batch, out_dim), jnp.float32),
        compiler_params=pltpu.CompilerParams(
            dimension_semantics=("arbitrary",),
        ),
    )(x, m0, m1, b2)
    return out
